# baseline (speedup 1.0000x reference)
import functools

import jax
import jax.numpy as jnp
from jax import lax
from jax.experimental import pallas as pl
from jax.experimental.pallas import tpu as pltpu
from jax.experimental.pallas import tpu_sc as plsc

D_MODEL = 128
MAX_REL = 128
SEQ_LEN = 1024
EREV_ROWS = 2 * SEQ_LEN

_SC_INFO = plsc.get_sparse_core_info()
_NC = _SC_INFO.num_cores
_NS = _SC_INFO.num_subcores
_NW = _NC * _NS
_LANES = _SC_INFO.num_lanes
_GATHER_PER_S = EREV_ROWS // _NS
_OUT_PER_W = SEQ_LEN // _NW


@functools.partial(
    pl.kernel,
    mesh=plsc.VectorSubcoreMesh(core_axis_name="c", subcore_axis_name="s"),
    out_type=jax.ShapeDtypeStruct((SEQ_LEN, SEQ_LEN, D_MODEL), jnp.float32),
    scratch_types=[
        pltpu.VMEM((_GATHER_PER_S,), jnp.int32),
        pltpu.VMEM((_GATHER_PER_S, D_MODEL), jnp.float32),
        pltpu.VMEM_SHARED((EREV_ROWS, D_MODEL), jnp.float32),
        pltpu.SemaphoreType.DMA,
    ],
)
def _sc_rel_emb(table_hbm, out_hbm, idx_v, rows_v, erev_sh, sem):
    c = lax.axis_index("c")
    s = lax.axis_index("s")
    base = s * _GATHER_PER_S
    lane = lax.iota(jnp.int32, _LANES)
    for v in range(_GATHER_PER_S // _LANES):
        k = lane + (base + v * _LANES)
        idx = jnp.clip((SEQ_LEN - 1) - k, -MAX_REL, MAX_REL) + MAX_REL
        idx_v[pl.ds(v * _LANES, _LANES)] = idx
    pltpu.async_copy(table_hbm.at[idx_v], rows_v, sem).wait()
    pltpu.sync_copy(rows_v, erev_sh.at[pl.ds(base, _GATHER_PER_S)])
    plsc.subcore_barrier()
    wid = s * _NC + c
    for r in range(_OUT_PER_W):
        i = wid * _OUT_PER_W + r
        start = (SEQ_LEN - 1) - i
        pltpu.sync_copy(erev_sh.at[pl.ds(start, SEQ_LEN)], out_hbm.at[i])


def kernel(seq_len, table):
    del seq_len
    return _sc_rel_emb(table)


# speedup vs baseline: 1.6326x; 1.6326x over previous
import functools

import jax
import jax.numpy as jnp
from jax import lax
from jax.experimental import pallas as pl
from jax.experimental.pallas import tpu as pltpu
from jax.experimental.pallas import tpu_sc as plsc

D_MODEL = 128
MAX_REL = 128
SEQ_LEN = 1024
EREV_ROWS = 2 * SEQ_LEN
ROWS_PER_STEP = 16

_NC = plsc.get_sparse_core_info().num_cores
_ROWS_PER_C = EREV_ROWS // _NC


@functools.partial(
    pl.kernel,
    mesh=plsc.ScalarSubcoreMesh(axis_name="c", num_cores=_NC),
    out_type=jax.ShapeDtypeStruct((EREV_ROWS, D_MODEL), jnp.float32),
    scratch_types=[pltpu.SemaphoreType.DMA],
)
def _sc_build_erev(table_hbm, erev_hbm, sem):
    base = lax.axis_index("c") * _ROWS_PER_C

    def _copy(k):
        idx = jnp.clip((SEQ_LEN - 1) - k, -MAX_REL, MAX_REL) + MAX_REL
        return pltpu.make_async_copy(table_hbm.at[idx], erev_hbm.at[k], sem)

    def _fire(r, carry):
        _copy(base + r).start()
        return carry

    def _drain(r, carry):
        _copy(base + r).wait()
        return carry

    lax.fori_loop(0, _ROWS_PER_C, _fire, 0)
    lax.fori_loop(0, _ROWS_PER_C, _drain, 0)


def _tc_body(erev_ref, out_ref):
    i = pl.program_id(0)
    for r in range(ROWS_PER_STEP):
        row = i * ROWS_PER_STEP + r
        start = (SEQ_LEN - 1) - row
        out_ref[r] = erev_ref[pl.ds(start, SEQ_LEN), :]


def kernel(seq_len, table):
    del seq_len
    erev = _sc_build_erev(table)
    return pl.pallas_call(
        _tc_body,
        grid=(SEQ_LEN // ROWS_PER_STEP,),
        in_specs=[pl.BlockSpec((EREV_ROWS, D_MODEL), lambda i: (0, 0))],
        out_specs=pl.BlockSpec((ROWS_PER_STEP, SEQ_LEN, D_MODEL),
                               lambda i: (i, 0, 0)),
        out_shape=jax.ShapeDtypeStruct((SEQ_LEN, SEQ_LEN, D_MODEL),
                                       jnp.float32),
    )(erev)
